# Initial kernel scaffold; baseline (speedup 1.0000x reference)
import jax
import jax.numpy as jnp
from jax.experimental import pallas as pl


def kernel(mu_in, Sigma_in):
    raise NotImplementedError("write your pallas kernel here")


# speedup vs baseline: 1.0108x; 1.0108x over previous
import functools

import jax
import jax.numpy as jnp
from jax.experimental import pallas as pl

_ROT_A = (13, 15, 26, 6)
_ROT_B = (17, 29, 16, 24)
_KS = (0, 42, 0x1BD11BDA ^ 42)
_KEEP_THRESH = 7549747
_INV_KEEP = float(1.0 / jnp.float32(0.9))
_COLS = 2048


def _rotl(x, r):
    return (x << jnp.uint32(r)) | (x >> jnp.uint32(32 - r))


def _threefry_keep_mask(flat_base, shape):
    row = jax.lax.broadcasted_iota(jnp.int32, shape, 0)
    col = jax.lax.broadcasted_iota(jnp.int32, shape, 1)
    x1 = (flat_base + row * _COLS + col).astype(jnp.uint32)
    x0 = jnp.zeros(shape, jnp.uint32)
    ks0, ks1, ks2 = (jnp.uint32(k) for k in _KS)
    x0 = x0 + ks0
    x1 = x1 + ks1
    ks = (ks0, ks1, ks2)
    for i in range(5):
        for r in (_ROT_A if i % 2 == 0 else _ROT_B):
            x0 = x0 + x1
            x1 = _rotl(x1, r)
            x1 = x1 ^ x0
        x0 = x0 + ks[(i + 1) % 3]
        x1 = x1 + ks[(i + 2) % 3] + jnp.uint32(i + 1)
    bits = x0 ^ x1
    return ((bits >> jnp.uint32(9)).astype(jnp.int32) < _KEEP_THRESH)


def _vdp_body(block_rows, mu_ref, sg_ref, muo_ref, sgo_ref):
    base = pl.program_id(0) * (block_rows * _COLS)
    keep = _threefry_keep_mask(base, mu_ref.shape)
    mu = mu_ref[...]
    zero = jnp.float32(0.0)
    muo_ref[...] = jnp.where(keep, mu * jnp.float32(_INV_KEEP), zero)
    nz = keep & (mu != zero)
    sgo_ref[...] = jnp.where(nz, sg_ref[...] * jnp.float32(1.0 / 2048.0), zero)


@functools.partial(jax.jit, static_argnames=("block_rows",))
def _vdp_flat(mu2, sg2, block_rows=256):
    rows = mu2.shape[0]
    grid = rows // block_rows
    spec = pl.BlockSpec((block_rows, _COLS), lambda i: (i, 0))
    out = pl.pallas_call(
        functools.partial(_vdp_body, block_rows),
        grid=(grid,),
        in_specs=[spec, spec],
        out_specs=[spec, spec],
        out_shape=[
            jax.ShapeDtypeStruct((rows, _COLS), jnp.float32),
            jax.ShapeDtypeStruct((rows, _COLS), jnp.float32),
        ],
    )(mu2, sg2)
    return out


def kernel(mu_in, Sigma_in):
    shape = mu_in.shape
    rows = shape[0] * shape[1]
    mu2 = mu_in.reshape(rows, _COLS)
    sg2 = Sigma_in.reshape(rows, _COLS)
    muo, sgo = _vdp_flat(mu2, sg2)
    return muo.reshape(shape), sgo.reshape(shape)
